# baseline (speedup 1.0000x reference)
import functools

import jax
import jax.numpy as jnp
from jax import lax
from jax.experimental import pallas as pl
from jax.experimental.pallas import tpu as pltpu
from jax.experimental.pallas import tpu_sc as plsc

N_FEATURES = 32768
D_MODEL = 2048
BATCH = 16384
K = 32
GRAD_CLIP_THRESHOLD = 1.0

NW = 32
PER_W = BATCH * K // NW
_UNROLL = 8


def _make_hist_kernel():
    mesh = plsc.VectorSubcoreMesh(core_axis_name="c", subcore_axis_name="s")

    @functools.partial(
        pl.kernel,
        out_type=jax.ShapeDtypeStruct((NW, N_FEATURES), jnp.float32),
        mesh=mesh,
        scratch_types=[
            pltpu.VMEM((PER_W,), jnp.int32),
            pltpu.VMEM((N_FEATURES,), jnp.float32),
        ],
        compiler_params=pltpu.CompilerParams(needs_layout_passes=False),
    )
    def hist_kernel(idx_hbm, out_hbm, idx_v, hist_v):
        wid = lax.axis_index("s") * 2 + lax.axis_index("c")
        zero = jnp.zeros((16,), jnp.float32)

        def zbody(i, c):
            for j in range(_UNROLL):
                hist_v[pl.ds((i * _UNROLL + j) * 16, 16)] = zero
            return c

        lax.fori_loop(0, N_FEATURES // (16 * _UNROLL), zbody, 0)

        pltpu.sync_copy(idx_hbm.at[pl.ds(wid * PER_W, PER_W)], idx_v)

        ones = jnp.ones((16,), jnp.float32)

        def body(i, c):
            for j in range(_UNROLL):
                vec = idx_v[pl.ds((i * _UNROLL + j) * 16, 16)]
                plsc.addupdate_scatter(hist_v, [vec], ones)
            return c

        lax.fori_loop(0, PER_W // (16 * _UNROLL), body, 0)

        pltpu.sync_copy(hist_v, out_hbm.at[wid])

    return hist_kernel


_X_BLOCK = 2048
_U_BLOCK = 1024
_GRID = BATCH // _X_BLOCK


def _reduce_body(x_ref, u_ref, norm_ref, clip_ref, acc_ref):
    i = pl.program_id(0)

    @pl.when(i == 0)
    def _init():
        acc_ref[0] = 0.0
        acc_ref[1] = 0.0

    xb = x_ref[...]
    rs = jnp.sum(xb * xb, axis=1, keepdims=True)
    nsum = jnp.sum(jnp.sqrt(rs))
    ub = u_ref[...]
    csum = jnp.sum((jnp.abs(ub) > GRAD_CLIP_THRESHOLD).astype(jnp.float32))
    acc_ref[0] += nsum
    acc_ref[1] += csum

    @pl.when(i == _GRID - 1)
    def _fini():
        norm_ref[0, 0] = acc_ref[0]
        clip_ref[0, 0] = acc_ref[1]


def _dense_reduce(x, updates_flat):
    return pl.pallas_call(
        _reduce_body,
        grid=(_GRID,),
        in_specs=[
            pl.BlockSpec((_X_BLOCK, D_MODEL), lambda i: (i, 0)),
            pl.BlockSpec((_U_BLOCK, D_MODEL), lambda i: (i, 0)),
        ],
        out_specs=[
            pl.BlockSpec((1, 1), lambda i: (0, 0), memory_space=pltpu.SMEM),
            pl.BlockSpec((1, 1), lambda i: (0, 0), memory_space=pltpu.SMEM),
        ],
        out_shape=[
            jax.ShapeDtypeStruct((1, 1), jnp.float32),
            jax.ShapeDtypeStruct((1, 1), jnp.float32),
        ],
        scratch_shapes=[pltpu.SMEM((2,), jnp.float32)],
        compiler_params=pltpu.CompilerParams(
            dimension_semantics=("arbitrary",)
        ),
    )(x, updates_flat)


def _blend_body(w_ref, nw_ref, fd_ref, h_ref, out_ref):
    h = jnp.sum(h_ref[...], axis=0)
    out_ref[...] = fd_ref[...] * w_ref[0] + h * nw_ref[0]


def _blend(w, nw, fd2, hists3):
    return pl.pallas_call(
        _blend_body,
        in_specs=[
            pl.BlockSpec(memory_space=pltpu.SMEM),
            pl.BlockSpec(memory_space=pltpu.SMEM),
            pl.BlockSpec(memory_space=pltpu.VMEM),
            pl.BlockSpec(memory_space=pltpu.VMEM),
        ],
        out_shape=jax.ShapeDtypeStruct(fd2.shape, jnp.float32),
    )(w, nw, fd2, hists3)


def kernel(n_steps, avg_norm, feature_density, grad_clip_percent, updates_flat, x, k_indices):
    ns = jnp.asarray(n_steps, jnp.float32)
    w = ns / (ns + 1.0)
    nw = 1.0 / (ns + 1.0)

    hist_kernel = _make_hist_kernel()
    hists = hist_kernel(k_indices.reshape(-1))

    norm_sum, clip_count = _dense_reduce(x, updates_flat)

    fd2 = feature_density.reshape(N_FEATURES // 128, 128)
    hists3 = hists.reshape(NW, N_FEATURES // 128, 128)
    updated_fd = _blend(
        w.reshape(1), nw.reshape(1), fd2, hists3
    ).reshape(N_FEATURES)

    new_avg_norm = norm_sum[0, 0] / BATCH
    updated_avg_norm = avg_norm * w + new_avg_norm * nw

    new_clip = clip_count[0, 0] / float(updates_flat.size)
    updated_clip = grad_clip_percent * w + new_clip * nw

    return (
        jnp.asarray(n_steps + 1),
        updated_avg_norm,
        updated_fd,
        updated_clip,
    )


# speedup vs baseline: 1.0133x; 1.0133x over previous
import functools

import jax
import jax.numpy as jnp
from jax import lax
from jax.experimental import pallas as pl
from jax.experimental.pallas import tpu as pltpu
from jax.experimental.pallas import tpu_sc as plsc

N_FEATURES = 32768
D_MODEL = 2048
BATCH = 16384
K = 32
GRAD_CLIP_THRESHOLD = 1.0

NW = 32
PER_W = BATCH * K // NW
_UNROLL = 8


def _make_hist_kernel():
    mesh = plsc.VectorSubcoreMesh(core_axis_name="c", subcore_axis_name="s")

    @functools.partial(
        pl.kernel,
        out_type=jax.ShapeDtypeStruct((NW, N_FEATURES), jnp.float32),
        mesh=mesh,
        scratch_types=[
            pltpu.VMEM((PER_W,), jnp.int32),
            pltpu.VMEM((N_FEATURES,), jnp.float32),
        ],
        compiler_params=pltpu.CompilerParams(needs_layout_passes=False, skip_device_barrier=True),
    )
    def hist_kernel(idx_hbm, out_hbm, idx_v, hist_v):
        wid = lax.axis_index("s") * 2 + lax.axis_index("c")
        zero = jnp.zeros((16,), jnp.float32)

        def zbody(i, c):
            for j in range(_UNROLL):
                hist_v[pl.ds((i * _UNROLL + j) * 16, 16)] = zero
            return c

        lax.fori_loop(0, N_FEATURES // (16 * _UNROLL), zbody, 0)

        pltpu.sync_copy(idx_hbm.at[pl.ds(wid * PER_W, PER_W)], idx_v)

        ones = jnp.ones((16,), jnp.float32)

        def body(i, c):
            for j in range(_UNROLL):
                vec = idx_v[pl.ds((i * _UNROLL + j) * 16, 16)]
                plsc.addupdate_scatter(hist_v, [vec], ones)
            return c

        lax.fori_loop(0, PER_W // (16 * _UNROLL), body, 0)

        pltpu.sync_copy(hist_v, out_hbm.at[wid])

    return hist_kernel


_X_BLOCK = 1024
_U_BLOCK = 512
_GRID = BATCH // _X_BLOCK


def _reduce_body(x_ref, u_ref, norm_ref, clip_ref, acc_ref):
    i = pl.program_id(0)

    @pl.when(i == 0)
    def _init():
        acc_ref[0] = 0.0
        acc_ref[1] = 0.0

    xb = x_ref[...]
    rs = jnp.sum(xb * xb, axis=1, keepdims=True)
    nsum = jnp.sum(jnp.sqrt(rs))
    ub = u_ref[...]
    csum = jnp.sum((jnp.abs(ub) > GRAD_CLIP_THRESHOLD).astype(jnp.float32))
    acc_ref[0] += nsum
    acc_ref[1] += csum

    @pl.when(i == _GRID - 1)
    def _fini():
        norm_ref[0, 0] = acc_ref[0]
        clip_ref[0, 0] = acc_ref[1]


def _dense_reduce(x, updates_flat):
    return pl.pallas_call(
        _reduce_body,
        grid=(_GRID,),
        in_specs=[
            pl.BlockSpec((_X_BLOCK, D_MODEL), lambda i: (i, 0)),
            pl.BlockSpec((_U_BLOCK, D_MODEL), lambda i: (i, 0)),
        ],
        out_specs=[
            pl.BlockSpec((1, 1), lambda i: (0, 0), memory_space=pltpu.SMEM),
            pl.BlockSpec((1, 1), lambda i: (0, 0), memory_space=pltpu.SMEM),
        ],
        out_shape=[
            jax.ShapeDtypeStruct((1, 1), jnp.float32),
            jax.ShapeDtypeStruct((1, 1), jnp.float32),
        ],
        scratch_shapes=[pltpu.SMEM((2,), jnp.float32)],
        compiler_params=pltpu.CompilerParams(
            dimension_semantics=("arbitrary",)
        ),
    )(x, updates_flat)


def _blend_body(w_ref, nw_ref, fd_ref, h_ref, out_ref):
    h = jnp.sum(h_ref[...], axis=0)
    out_ref[...] = fd_ref[...] * w_ref[0] + h * nw_ref[0]


def _blend(w, nw, fd2, hists3):
    return pl.pallas_call(
        _blend_body,
        in_specs=[
            pl.BlockSpec(memory_space=pltpu.SMEM),
            pl.BlockSpec(memory_space=pltpu.SMEM),
            pl.BlockSpec(memory_space=pltpu.VMEM),
            pl.BlockSpec(memory_space=pltpu.VMEM),
        ],
        out_shape=jax.ShapeDtypeStruct(fd2.shape, jnp.float32),
    )(w, nw, fd2, hists3)


def kernel(n_steps, avg_norm, feature_density, grad_clip_percent, updates_flat, x, k_indices):
    ns = jnp.asarray(n_steps, jnp.float32)
    w = ns / (ns + 1.0)
    nw = 1.0 / (ns + 1.0)

    hist_kernel = _make_hist_kernel()
    hists = hist_kernel(k_indices.reshape(-1))

    norm_sum, clip_count = _dense_reduce(x, updates_flat)

    fd2 = feature_density.reshape(N_FEATURES // 128, 128)
    hists3 = hists.reshape(NW, N_FEATURES // 128, 128)
    updated_fd = _blend(
        w.reshape(1), nw.reshape(1), fd2, hists3
    ).reshape(N_FEATURES)

    new_avg_norm = norm_sum[0, 0] / BATCH
    updated_avg_norm = avg_norm * w + new_avg_norm * nw

    new_clip = clip_count[0, 0] / float(updates_flat.size)
    updated_clip = grad_clip_percent * w + new_clip * nw

    return (
        jnp.asarray(n_steps + 1),
        updated_avg_norm,
        updated_fd,
        updated_clip,
    )


# speedup vs baseline: 1.0753x; 1.0612x over previous
import functools

import jax
import jax.numpy as jnp
from jax import lax
from jax.experimental import pallas as pl
from jax.experimental.pallas import tpu as pltpu
from jax.experimental.pallas import tpu_sc as plsc

N_FEATURES = 32768
D_MODEL = 2048
BATCH = 16384
K = 32
GRAD_CLIP_THRESHOLD = 1.0

NW = 32
PER_W = BATCH * K // NW
ROWS_W = BATCH // NW
_UNROLL = 8


def _make_hist_kernel():
    mesh = plsc.VectorSubcoreMesh(core_axis_name="c", subcore_axis_name="s")

    @functools.partial(
        pl.kernel,
        out_type=jax.ShapeDtypeStruct((NW, N_FEATURES), jnp.float32),
        mesh=mesh,
        scratch_types=[
            pltpu.VMEM((ROWS_W, K), jnp.int32),
            pltpu.VMEM((N_FEATURES,), jnp.float32),
        ],
        compiler_params=pltpu.CompilerParams(needs_layout_passes=False, skip_device_barrier=True),
    )
    def hist_kernel(idx_hbm, out_hbm, idx_v, hist_v):
        wid = lax.axis_index("s") * 2 + lax.axis_index("c")
        zero = jnp.zeros((16,), jnp.float32)

        def zbody(i, c):
            for j in range(_UNROLL):
                hist_v[pl.ds((i * _UNROLL + j) * 16, 16)] = zero
            return c

        lax.fori_loop(0, N_FEATURES // (16 * _UNROLL), zbody, 0)

        pltpu.sync_copy(idx_hbm.at[pl.ds(wid * ROWS_W, ROWS_W), :], idx_v)

        ones = jnp.ones((16,), jnp.float32)

        def body(i, c):
            for r in range(4):
                for j in range(K // 16):
                    vec = idx_v[i * 4 + r, pl.ds(j * 16, 16)]
                    plsc.addupdate_scatter(hist_v, [vec], ones)
            return c

        lax.fori_loop(0, ROWS_W // 4, body, 0)

        pltpu.sync_copy(hist_v, out_hbm.at[wid])

    return hist_kernel


_X_BLOCK = 1024
_U_BLOCK = 512
_GRID = BATCH // _X_BLOCK


def _reduce_body(x_ref, u_ref, norm_ref, clip_ref, acc_ref):
    i = pl.program_id(0)

    @pl.when(i == 0)
    def _init():
        acc_ref[0] = 0.0
        acc_ref[1] = 0.0

    xb = x_ref[...]
    rs = jnp.sum(xb * xb, axis=1, keepdims=True)
    nsum = jnp.sum(jnp.sqrt(rs))
    ub = u_ref[...]
    csum = jnp.sum((jnp.abs(ub) > GRAD_CLIP_THRESHOLD).astype(jnp.float32))
    acc_ref[0] += nsum
    acc_ref[1] += csum

    @pl.when(i == _GRID - 1)
    def _fini():
        norm_ref[0, 0] = acc_ref[0]
        clip_ref[0, 0] = acc_ref[1]


def _dense_reduce(x, updates_flat):
    return pl.pallas_call(
        _reduce_body,
        grid=(_GRID,),
        in_specs=[
            pl.BlockSpec((_X_BLOCK, D_MODEL), lambda i: (i, 0)),
            pl.BlockSpec((_U_BLOCK, D_MODEL), lambda i: (i, 0)),
        ],
        out_specs=[
            pl.BlockSpec((1, 1), lambda i: (0, 0), memory_space=pltpu.SMEM),
            pl.BlockSpec((1, 1), lambda i: (0, 0), memory_space=pltpu.SMEM),
        ],
        out_shape=[
            jax.ShapeDtypeStruct((1, 1), jnp.float32),
            jax.ShapeDtypeStruct((1, 1), jnp.float32),
        ],
        scratch_shapes=[pltpu.SMEM((2,), jnp.float32)],
        compiler_params=pltpu.CompilerParams(
            dimension_semantics=("arbitrary",)
        ),
    )(x, updates_flat)


def _blend_body(w_ref, nw_ref, fd_ref, h_ref, out_ref):
    h = jnp.sum(h_ref[...], axis=0)
    out_ref[...] = fd_ref[...] * w_ref[0] + h * nw_ref[0]


def _blend(w, nw, fd2, hists3):
    return pl.pallas_call(
        _blend_body,
        in_specs=[
            pl.BlockSpec(memory_space=pltpu.SMEM),
            pl.BlockSpec(memory_space=pltpu.SMEM),
            pl.BlockSpec(memory_space=pltpu.VMEM),
            pl.BlockSpec(memory_space=pltpu.VMEM),
        ],
        out_shape=jax.ShapeDtypeStruct(fd2.shape, jnp.float32),
    )(w, nw, fd2, hists3)


def kernel(n_steps, avg_norm, feature_density, grad_clip_percent, updates_flat, x, k_indices):
    ns = jnp.asarray(n_steps, jnp.float32)
    w = ns / (ns + 1.0)
    nw = 1.0 / (ns + 1.0)

    hist_kernel = _make_hist_kernel()
    hists = hist_kernel(k_indices)

    norm_sum, clip_count = _dense_reduce(x, updates_flat)

    fd2 = feature_density.reshape(N_FEATURES // 128, 128)
    hists3 = hists.reshape(NW, N_FEATURES // 128, 128)
    updated_fd = _blend(
        w.reshape(1), nw.reshape(1), fd2, hists3
    ).reshape(N_FEATURES)

    new_avg_norm = norm_sum[0, 0] / BATCH
    updated_avg_norm = avg_norm * w + new_avg_norm * nw

    new_clip = clip_count[0, 0] / float(updates_flat.size)
    updated_clip = grad_clip_percent * w + new_clip * nw

    return (
        jnp.asarray(n_steps + 1),
        updated_avg_norm,
        updated_fd,
        updated_clip,
    )
